# baseline (speedup 1.0000x reference)
import functools

import jax
import jax.numpy as jnp
from jax.experimental import pallas as pl
from jax.experimental.pallas import tpu as pltpu

BATCH = 4096
D = 256
K = 1024
L = 4
TB = 512


def _trunc16(c):
    bits = jax.lax.bitcast_convert_type(c, jnp.uint32)
    return jax.lax.bitcast_convert_type(
        jnp.bitwise_and(bits, jnp.uint32(0xFFFF0000)), jnp.float32)


def _body(f_ref, wt_ref, b_ref, cb_ref, cbt_ref, logits_ref, qsum_ref,
          hi_ref, mid_ref, lo_ref, cn_ref):
    f32 = jnp.float32

    @pl.when(pl.program_id(0) == 0)
    def _init():
        c = cb_ref[...]
        hi = _trunc16(c)
        r1 = c - hi
        mid = _trunc16(r1)
        hi_ref[...] = hi
        mid_ref[...] = mid
        lo_ref[...] = r1 - mid
        for l in range(L):
            cbt_l = cbt_ref[l]
            cn_ref[l] = jnp.sum(cbt_l * cbt_l, axis=0, keepdims=True)

    f = f_ref[...]
    x = jax.lax.dot_general(f, wt_ref[...], (((1,), (0,)), ((), ())),
                            preferred_element_type=f32)
    res = x + b_ref[...]
    qsum = jnp.zeros((TB, D), f32)
    for l in range(L):
        cbt_l = cbt_ref[l]
        g = jax.lax.dot_general(res, cbt_l, (((1,), (0,)), ((), ())),
                                preferred_element_type=f32)
        rn = jnp.sum(res * res, axis=1, keepdims=True)
        d2 = rn + cn_ref[l] - 2.0 * g
        dist = jnp.sqrt(jnp.maximum(d2, 1e-12))
        logits_ref[:, l, :] = -dist
        ids = jnp.argmin(d2, axis=1, keepdims=True)
        iota = jax.lax.broadcasted_iota(jnp.int32, (TB, K), 1)
        onehot = (iota == ids).astype(f32)
        dn = (((1,), (0,)), ((), ()))
        q = ((jax.lax.dot_general(onehot, hi_ref[l], dn,
                                  preferred_element_type=f32)
              + jax.lax.dot_general(onehot, mid_ref[l], dn,
                                    preferred_element_type=f32))
             + jax.lax.dot_general(onehot, lo_ref[l], dn,
                                   preferred_element_type=f32))
        qsum = qsum + q
        res = res - q
    qsum_ref[...] = qsum


@functools.partial(jax.jit, static_argnames=("interpret",))
def kernel(features, W_proj, b_proj, codebooks, interpret=False):
    wt = jnp.swapaxes(W_proj, 0, 1)
    cbt = jnp.swapaxes(codebooks, 1, 2)
    b2 = b_proj.reshape(1, D)
    grid = (BATCH // TB,)
    logits2d, qsum = pl.pallas_call(
        _body,
        grid=grid,
        in_specs=[
            pl.BlockSpec((TB, D), lambda i: (i, 0)),
            pl.BlockSpec((D, D), lambda i: (0, 0)),
            pl.BlockSpec((1, D), lambda i: (0, 0)),
            pl.BlockSpec((L, K, D), lambda i: (0, 0, 0)),
            pl.BlockSpec((L, D, K), lambda i: (0, 0, 0)),
        ],
        out_specs=[
            pl.BlockSpec((TB, L, K), lambda i: (i, 0, 0)),
            pl.BlockSpec((TB, D), lambda i: (i, 0)),
        ],
        out_shape=[
            jax.ShapeDtypeStruct((BATCH, L, K), jnp.float32),
            jax.ShapeDtypeStruct((BATCH, D), jnp.float32),
        ],
        scratch_shapes=[
            pltpu.VMEM((L, K, D), jnp.float32),
            pltpu.VMEM((L, K, D), jnp.float32),
            pltpu.VMEM((L, K, D), jnp.float32),
            pltpu.VMEM((L, 1, K), jnp.float32),
        ],
        interpret=interpret,
    )(features, wt, b2, codebooks, cbt)
    return logits2d, qsum
